# Initial kernel scaffold; baseline (speedup 1.0000x reference)
import jax
import jax.numpy as jnp
from jax.experimental import pallas as pl


def kernel(x, edge_index, bbox, W1_rel, b1, W1_root, W2_rel, b2, W2_root):
    raise NotImplementedError("write your pallas kernel here")


# speedup vs baseline: 4.0659x; 4.0659x over previous
import functools

import jax
import jax.numpy as jnp
from jax import lax
from jax.experimental import pallas as pl
from jax.experimental.pallas import tpu as pltpu
from jax.experimental.pallas import tpu_sc as plsc

NC = 2
NS = 16
NW = NC * NS
CHUNK = 128


def _leaky(y):
    return jnp.where(y >= 0, y, 0.01 * y)


@functools.lru_cache(maxsize=None)
def _make_segsum(n_nodes, d, e_pad, gpad):
    kpt = e_pad // (NW * CHUNK)
    n_acc = ((n_nodes + 1 + 8 * NS - 1) // (8 * NS)) * (8 * NS)
    zrows = n_acc // NS
    mesh = plsc.VectorSubcoreMesh(core_axis_name="c", subcore_axis_name="s")

    @functools.partial(
        pl.kernel,
        out_type=[
            jax.ShapeDtypeStruct((NC, n_acc, d), jnp.float32),
            jax.ShapeDtypeStruct((gpad, d), jnp.float32),
        ],
        mesh=mesh,
        scratch_types=[
            pltpu.VMEM((CHUNK,), jnp.int32),
            pltpu.VMEM((CHUNK,), jnp.int32),
            pltpu.VMEM((CHUNK, d), jnp.float32),
            pltpu.VMEM_SHARED((n_acc, d), jnp.float32),
            pltpu.SemaphoreType.DMA,
            pltpu.VMEM((gpad,), jnp.int32),
            pltpu.VMEM((gpad, d), jnp.float32),
        ],
    )
    def segsum(src_hbm, dst_hbm, feat_hbm, zeros_hbm, bbox_hbm,
               out_hbm, gfeat_hbm,
               idx_s, idx_d, rows, acc, sem, bidx, brows):
        c = lax.axis_index("c")
        s = lax.axis_index("s")
        wid = s * NC + c

        pltpu.sync_copy(zeros_hbm, acc.at[pl.ds(s * zrows, zrows)])
        plsc.subcore_barrier()

        def body(k, carry):
            base = (wid * kpt + k) * CHUNK
            pltpu.sync_copy(src_hbm.at[pl.ds(base, CHUNK)], idx_s)
            gcp = pltpu.async_copy(feat_hbm.at[idx_s], rows, sem)
            pltpu.sync_copy(dst_hbm.at[pl.ds(base, CHUNK)], idx_d)
            gcp.wait()
            pltpu.sync_copy(rows, acc.at[idx_d], add=True)
            return carry

        lax.fori_loop(0, kpt, body, 0)
        plsc.subcore_barrier()
        pltpu.sync_copy(acc.at[pl.ds(s * zrows, zrows)],
                        out_hbm.at[c, pl.ds(s * zrows, zrows)])

        @pl.when(wid == 0)
        def _():
            pltpu.sync_copy(bbox_hbm, bidx)
            pltpu.async_copy(feat_hbm.at[bidx], brows, sem).wait()
            pltpu.sync_copy(brows, gfeat_hbm)

    return segsum


@functools.lru_cache(maxsize=None)
def _make_gatherq(n_nodes, d, gpad):
    mesh = plsc.VectorSubcoreMesh(core_axis_name="c", subcore_axis_name="s")

    @functools.partial(
        pl.kernel,
        out_type=jax.ShapeDtypeStruct((NC, gpad, d), jnp.float32),
        mesh=mesh,
        scratch_types=[
            pltpu.VMEM((gpad,), jnp.int32),
            pltpu.VMEM((gpad, d), jnp.float32),
            pltpu.SemaphoreType.DMA,
        ],
    )
    def gatherq(q_hbm, bbox_hbm, out_hbm, bidx, brows, sem):
        c = lax.axis_index("c")
        s = lax.axis_index("s")
        wid = s * NC + c

        @pl.when(wid < NC)
        def _():
            pltpu.sync_copy(bbox_hbm, bidx)
            pltpu.async_copy(q_hbm.at[wid].at[bidx], brows, sem).wait()
            pltpu.sync_copy(brows, out_hbm.at[wid])

    return gatherq


def _tc1_body(p_ref, x_ref, wrel_ref, b_ref, wroot_ref, o_ref):
    agg = p_ref[0] + p_ref[1]
    y = lax.dot_general(agg, wrel_ref[...], (((1,), (1,)), ((), ())),
                        preferred_element_type=jnp.float32)
    y = y + b_ref[...] + lax.dot_general(
        x_ref[...], wroot_ref[...], (((1,), (1,)), ((), ())),
        preferred_element_type=jnp.float32)
    o_ref[...] = _leaky(y)


def _tc2_body(gq_ref, gh_ref, wrel_ref, b_ref, wroot_ref, o_ref):
    agg = gq_ref[0] + gq_ref[1]
    y = lax.dot_general(agg, wrel_ref[...], (((1,), (1,)), ((), ())),
                        preferred_element_type=jnp.float32)
    y = y + b_ref[...] + lax.dot_general(
        gh_ref[...], wroot_ref[...], (((1,), (1,)), ((), ())),
        preferred_element_type=jnp.float32)
    o_ref[...] = _leaky(y)


def kernel(x, edge_index, bbox, W1_rel, b1, W1_root, W2_rel, b2, W2_root):
    n, d = x.shape
    e = edge_index.shape[1]
    r = bbox.shape[0]

    e_pad = ((e + NW * CHUNK - 1) // (NW * CHUNK)) * (NW * CHUNK)
    gpad = ((r + 7) // 8) * 8

    src = edge_index[0]
    dst = edge_index[1]
    src_p = jnp.concatenate([src, jnp.zeros((e_pad - e,), jnp.int32)])
    dst_p = jnp.concatenate([dst, jnp.full((e_pad - e,), n, jnp.int32)])
    bbox_p = jnp.concatenate([bbox, jnp.zeros((gpad - r,), jnp.int32)])

    n_acc = ((n + 1 + 8 * NS - 1) // (8 * NS)) * (8 * NS)
    zeros_hbm = jnp.zeros((n_acc // NS, d), jnp.float32)
    b1_2d = b1.reshape(1, d)
    b2_2d = b2.reshape(1, d)

    segsum = _make_segsum(n, d, e_pad, gpad)
    gatherq = _make_gatherq(n, d, gpad)

    p1, _ = segsum(src_p, dst_p, x, zeros_hbm, bbox_p)

    rb = 2000
    h = pl.pallas_call(
        _tc1_body,
        grid=(n // rb,),
        in_specs=[
            pl.BlockSpec((NC, rb, d), lambda i: (0, i, 0)),
            pl.BlockSpec((rb, d), lambda i: (i, 0)),
            pl.BlockSpec((d, d), lambda i: (0, 0)),
            pl.BlockSpec((1, d), lambda i: (0, 0)),
            pl.BlockSpec((d, d), lambda i: (0, 0)),
        ],
        out_specs=pl.BlockSpec((rb, d), lambda i: (i, 0)),
        out_shape=jax.ShapeDtypeStruct((n, d), jnp.float32),
    )(p1, x, W1_rel, b1_2d, W1_root)

    p2, gh = segsum(src_p, dst_p, h, zeros_hbm, bbox_p)

    gq = gatherq(p2, bbox_p)

    out = pl.pallas_call(
        _tc2_body,
        out_shape=jax.ShapeDtypeStruct((gpad, d), jnp.float32),
    )(gq, gh, W2_rel, b2_2d, W2_root)

    return out[:r]
